# baseline (speedup 1.0000x reference)
import functools

import jax
import jax.numpy as jnp
from jax import lax
from jax.experimental import pallas as pl
from jax.experimental.pallas import tpu as pltpu
from jax.experimental.pallas import tpu_sc as plsc

N = 100000
D_N = 64
D_C = 32
D_OUT = D_N + D_C
ROW = 128
V_C = 1000
NW = 32
CHUNK = 128
K = 25
NBN = 5
NBC = 2
B_W = CHUNK * K
LAST_BASE = N - B_W


def kernel(node_type, charge_state, W_node_type, W_charge_state):
    wn_pack = jnp.concatenate(
        [W_node_type[0::2], W_node_type[1::2]], axis=1)
    wc_pad = jnp.pad(W_charge_state, ((0, 0), (0, ROW - D_C)))
    idx_half = jax.lax.shift_right_logical(node_type, 1)

    mesh = plsc.VectorSubcoreMesh(core_axis_name="c", subcore_axis_name="s")

    @functools.partial(
        pl.kernel,
        mesh=mesh,
        out_type=(
            jax.ShapeDtypeStruct((N, ROW), jnp.float32),
            jax.ShapeDtypeStruct((N, ROW), jnp.float32),
        ),
        scratch_types=[
            pltpu.VMEM((B_W,), jnp.int32),
            pltpu.VMEM((B_W,), jnp.int32),
            pltpu.VMEM_SHARED((V_C, ROW), jnp.float32),
        ]
        + [pltpu.VMEM((CHUNK, ROW), jnp.float32) for _ in range(NBN + NBC)]
        + [pltpu.SemaphoreType.DMA] * (1 + 2 * NBN + 2 * NBC),
    )
    def body(nt_hbm, cs_hbm, wn_hbm, wc_hbm, outn_hbm, outc_hbm,
             idxn_v, idxc_v, wc_sh, *rest):
        rn_v = rest[:NBN]
        rc_v = rest[NBN:NBN + NBC]
        sems = rest[NBN + NBC:]
        sem_i = sems[0]
        sem_n = sems[1:NBN + 1]
        sem_wn = sems[NBN + 1:2 * NBN + 1]
        sem_c = sems[2 * NBN + 1:2 * NBN + 1 + NBC]
        sem_wc = sems[2 * NBN + 1 + NBC:2 * NBN + 1 + 2 * NBC]

        cid = lax.axis_index("c")
        sid = lax.axis_index("s")
        wid = sid * 2 + cid
        base = jnp.where(wid == NW - 1, LAST_BASE, wid * B_W)

        cpn = pltpu.async_copy(nt_hbm.at[pl.ds(base, B_W)], idxn_v, sem_i)
        cpc = pltpu.async_copy(cs_hbm.at[pl.ds(base, B_W)], idxc_v, sem_i)

        @pl.when(sid == 0)
        def _():
            pltpu.sync_copy(wc_hbm, wc_sh)

        cpn.wait()
        cpc.wait()

        def gn(j, b):
            pltpu.async_copy(
                wn_hbm.at[idxn_v.at[pl.ds(j * CHUNK, CHUNK)]],
                rn_v[b], sem_n[b])

        def gc(j, b):
            pltpu.async_copy(
                wc_sh.at[idxc_v.at[pl.ds(j * CHUNK, CHUNK)]],
                rc_v[b], sem_c[b])

        def wn(j, b):
            row0 = base + j * CHUNK
            pltpu.async_copy(rn_v[b], outn_hbm.at[pl.ds(row0, CHUNK)],
                             sem_wn[b])

        def wc(j, b):
            row0 = base + j * CHUNK
            pltpu.async_copy(rc_v[b], outc_hbm.at[pl.ds(row0, CHUNK)],
                             sem_wc[b])

        def drain(sem_b):
            pltpu.make_async_copy(
                wn_hbm.at[pl.ds(0, CHUNK)], rn_v[0], sem_b).wait()

        gn(0, 0)
        gn(1, 1)
        gn(2, 2)
        plsc.subcore_barrier()
        gc(0, 0)

        def chunk_body(i, ip, dn_w, dc_w, gn_on, gc_on):
            b, cb = ip % NBN, ip % NBC
            if gn_on:
                if dn_w:
                    drain(sem_wn[(ip + 3) % NBN])
                gn(i + 3, (ip + 3) % NBN)
            if gc_on:
                if dc_w:
                    drain(sem_wc[(ip + 1) % NBC])
                gc(i + 1, (ip + 1) % NBC)
            drain(sem_n[b])
            wn(i, b)
            drain(sem_c[cb])
            wc(i, cb)

        chunk_body(0, 0, dn_w=False, dc_w=False, gn_on=True, gc_on=True)
        chunk_body(1, 1, dn_w=False, dc_w=True, gn_on=True, gc_on=True)

        def group(g, carry):
            for b2 in range(10):
                i = 2 + g * 10 + b2
                chunk_body(i, 2 + b2, dn_w=True, dc_w=True,
                           gn_on=True, gc_on=True)
            return carry

        lax.fori_loop(0, 2, group, 0)

        chunk_body(22, 22, dn_w=True, dc_w=True, gn_on=False, gc_on=True)
        chunk_body(23, 23, dn_w=True, dc_w=True, gn_on=False, gc_on=True)
        chunk_body(24, 24, dn_w=True, dc_w=True, gn_on=False, gc_on=False)

        for b in range(NBN):
            drain(sem_wn[b])
        drain(sem_wc[23 % NBC])
        drain(sem_wc[24 % NBC])

    outn, outc = body(idx_half, charge_state, wn_pack, wc_pad)
    par = (node_type & 1)[:, None]
    node_part = jnp.where(par == 1, outn[:, D_N:2 * D_N], outn[:, :D_N])
    return jnp.concatenate([node_part, outc[:, :D_C]], axis=-1)


# speedup vs baseline: 7.3291x; 7.3291x over previous
import functools

import jax
import jax.numpy as jnp
from jax import lax
from jax.experimental import pallas as pl
from jax.experimental.pallas import tpu as pltpu
from jax.experimental.pallas import tpu_sc as plsc

N = 100000
D_N = 64
D_C = 32
D_OUT = D_N + D_C
ROW = 128
V_C = 1000
NW = 32
CHUNK = 128
K = 25
NBUF = 5
B_W = CHUNK * K
LAST_BASE = N - B_W


def kernel(node_type, charge_state, W_node_type, W_charge_state):
    wn_pad = jnp.pad(W_node_type, ((0, 0), (0, ROW - D_N)))
    wc_pad = jnp.pad(W_charge_state, ((0, 0), (D_N, ROW - D_OUT)))

    mesh = plsc.VectorSubcoreMesh(core_axis_name="c", subcore_axis_name="s")

    @functools.partial(
        pl.kernel,
        mesh=mesh,
        out_type=jax.ShapeDtypeStruct((N, ROW), jnp.float32),
        scratch_types=[
            pltpu.VMEM((B_W,), jnp.int32),
            pltpu.VMEM((B_W,), jnp.int32),
            pltpu.VMEM_SHARED((V_C, ROW), jnp.float32),
        ]
        + [pltpu.VMEM((CHUNK, ROW), jnp.float32) for _ in range(NBUF)]
        + [pltpu.SemaphoreType.DMA] * (1 + 3 * NBUF),
    )
    def body(nt_hbm, cs_hbm, wn_hbm, wc_hbm, out_hbm,
             idxn_v, idxc_v, wc_sh, *rest):
        r_v = rest[:NBUF]
        sem_i = rest[NBUF]
        sem_n = rest[NBUF + 1:2 * NBUF + 1]
        sem_c = rest[2 * NBUF + 1:3 * NBUF + 1]
        sem_w = rest[3 * NBUF + 1:4 * NBUF + 1]

        cid = lax.axis_index("c")
        sid = lax.axis_index("s")
        wid = sid * 2 + cid
        base = jnp.where(wid == NW - 1, LAST_BASE, wid * B_W)

        cpn = pltpu.async_copy(nt_hbm.at[pl.ds(base, B_W)], idxn_v, sem_i)
        cpc = pltpu.async_copy(cs_hbm.at[pl.ds(base, B_W)], idxc_v, sem_i)

        @pl.when(sid == 0)
        def _():
            pltpu.sync_copy(wc_hbm, wc_sh)

        cpn.wait()
        cpc.wait()

        def gather(j, b):
            pltpu.async_copy(
                wn_hbm.at[idxn_v.at[pl.ds(j * CHUNK, CHUNK)]], r_v[b], sem_n[b])

        def charge_add(j, b):
            return pltpu.async_copy(
                wc_sh.at[idxc_v.at[pl.ds(j * CHUNK, CHUNK)]], r_v[b],
                sem_c[b], add=True)

        def write(j, b):
            row0 = base + j * CHUNK
            pltpu.async_copy(r_v[b], out_hbm.at[pl.ds(row0, CHUNK)], sem_w[b])

        def drain(sem_b):
            pltpu.make_async_copy(
                wn_hbm.at[pl.ds(0, CHUNK)], r_v[0], sem_b).wait()

        gather(0, 0)
        gather(1, 1)
        gather(2, 2)
        plsc.subcore_barrier()

        for i in (0, 1):
            b = i % NBUF
            gather(i + 3, (b + 3) % NBUF)
            drain(sem_n[b])
            charge_add(i, b).wait()
            write(i, b)

        def group(g, carry):
            for b2 in range(NBUF):
                i = 2 + g * NBUF + b2
                b = (2 + b2) % NBUF
                bg = (b + 3) % NBUF
                drain(sem_w[bg])
                gather(i + 3, bg)
                drain(sem_n[b])
                charge_add(i, b).wait()
                write(i, b)
            return carry

        lax.fori_loop(0, 4, group, 0)

        for i in (22, 23, 24):
            b = i % NBUF
            drain(sem_n[b])
            charge_add(i, b).wait()
            write(i, b)

        for b in range(NBUF):
            drain(sem_w[b])

    out128 = body(node_type, charge_state, wn_pad, wc_pad)
    return out128[:, :D_OUT]
